# Initial kernel scaffold; baseline (speedup 1.0000x reference)
import jax
import jax.numpy as jnp
from jax.experimental import pallas as pl


def kernel(x, centers, widths, consequents, rules):
    raise NotImplementedError("write your pallas kernel here")


# speedup vs baseline: 7471.3409x; 7471.3409x over previous
import functools

import jax
import jax.numpy as jnp
from jax import lax
from jax.experimental import pallas as pl
from jax.experimental.pallas import tpu as pltpu

BATCH = 1024
D = 26
M = 4
R = 2048
C = 16
DM = D * M
JC = (D + 1) * C
K = max(1, int(0.1 * R))

BB = 256
GRID = BATCH // BB


def _body(x_ref, xr_ref, cf_ref, wf_ref, rules_ref, cons_ref,
          out_ref, norm_ref, mask_ref):
    xr = xr_ref[...]
    cf = cf_ref[...]
    wf = wf_ref[...]
    inv = 1.0 / (2.0 * wf * wf)
    dq = xr - cf
    q = dq * dq * inv

    mcol = lax.broadcasted_iota(jnp.int32, (R, DM), 1) % M
    S = (rules_ref[...] == mcol).astype(jnp.float32)

    logits = lax.dot_general(q, S, (((1,), (1,)), ((), ())),
                             preferred_element_type=jnp.float32,
                             precision=lax.Precision.HIGHEST)
    firing = jnp.exp(-logits)
    fi = lax.bitcast_convert_type(firing, jnp.int32)

    lo0 = jnp.zeros((BB, 1), jnp.int32)
    hi0 = jnp.full((BB, 1), 0x40000000, jnp.int32)

    def step(_, carry):
        lo, hi = carry
        mid = (lo + hi) >> 1
        cnt = jnp.sum((fi >= mid).astype(jnp.int32), axis=1, keepdims=True)
        ge = cnt >= K
        return (jnp.where(ge, mid, lo), jnp.where(ge, hi, mid))

    lo, hi = lax.fori_loop(0, 30, step, (lo0, hi0))
    kth = lo

    maskf = (fi >= kth).astype(jnp.float32)
    fm = firing * maskf
    denom = jnp.sum(fm, axis=1, keepdims=True) + 1e-9
    normv = fm / denom

    P = (lax.broadcasted_iota(jnp.int32, (JC, C), 0) % C
         == lax.broadcasted_iota(jnp.int32, (JC, C), 1)).astype(jnp.float32)
    Cs = lax.dot_general(cons_ref[...], P, (((1,), (0,)), ((), ())),
                         preferred_element_type=jnp.float32,
                         precision=lax.Precision.HIGHEST)

    s_ext = jnp.sum(x_ref[...], axis=1, keepdims=True) + 1.0
    outv = s_ext * lax.dot_general(normv, Cs, (((1,), (0,)), ((), ())),
                                   preferred_element_type=jnp.float32,
                                   precision=lax.Precision.HIGHEST)

    out_ref[...] = outv
    norm_ref[...] = normv
    mask_ref[...] = maskf


@functools.partial(jax.jit, static_argnames=("interpret",))
def kernel(x, centers, widths, consequents, rules, interpret=False):
    x = x.astype(jnp.float32)
    xr = jnp.repeat(x, M, axis=1)
    cf = centers.astype(jnp.float32).reshape(1, DM)
    wf = widths.astype(jnp.float32).reshape(1, DM)
    rules_rep = jnp.repeat(rules.astype(jnp.int32), M, axis=1)
    cons2 = consequents.astype(jnp.float32).reshape(R, JC)

    out, norm, mask = pl.pallas_call(
        _body,
        grid=(GRID,),
        in_specs=[
            pl.BlockSpec((BB, D), lambda i: (i, 0)),
            pl.BlockSpec((BB, DM), lambda i: (i, 0)),
            pl.BlockSpec((1, DM), lambda i: (0, 0)),
            pl.BlockSpec((1, DM), lambda i: (0, 0)),
            pl.BlockSpec((R, DM), lambda i: (0, 0)),
            pl.BlockSpec((R, JC), lambda i: (0, 0)),
        ],
        out_specs=[
            pl.BlockSpec((BB, C), lambda i: (i, 0)),
            pl.BlockSpec((BB, R), lambda i: (i, 0)),
            pl.BlockSpec((BB, R), lambda i: (i, 0)),
        ],
        out_shape=[
            jax.ShapeDtypeStruct((BATCH, C), jnp.float32),
            jax.ShapeDtypeStruct((BATCH, R), jnp.float32),
            jax.ShapeDtypeStruct((BATCH, R), jnp.float32),
        ],
        compiler_params=pltpu.CompilerParams(
            dimension_semantics=("parallel",),
        ),
        interpret=interpret,
    )(x, xr, cf, wf, rules_rep, cons2)
    return (out, norm, mask)
